# Initial kernel scaffold; baseline (speedup 1.0000x reference)
import jax
import jax.numpy as jnp
from jax.experimental import pallas as pl


def kernel(x, sel_l0, sel_l1, sel_l2, sel_l3):
    raise NotImplementedError("write your pallas kernel here")


# speedup vs baseline: 1.8233x; 1.8233x over previous
import functools
import jax
import jax.numpy as jnp
from jax.experimental import pallas as pl
from jax.experimental.pallas import tpu as pltpu

_NB = 8


def _layer(h, sel_ref, out_dim, is_min):
    off = jnp.float32(2.0) if is_min else jnp.float32(-1.0)
    cols = []
    for j0 in range(0, out_dim, 8):
        sel = sel_ref[j0:j0 + 8, :]
        v = jnp.where((sel == 1)[None], h[:, None, :], off)
        r = v.min(axis=-1) if is_min else v.max(axis=-1)
        cols.append(r)
    return jnp.concatenate(cols, axis=1)


def _tc_body(x_ref, s0_ref, s1_ref, s2_ref, s3_ref, o_ref):
    h = x_ref[...]
    h = _layer(h, s0_ref, 256, True)
    h = _layer(h, s1_ref, 128, False)
    h = _layer(h, s2_ref, 256, True)
    o_ref[...] = _layer(h, s3_ref, 512, False)


@jax.jit
def kernel(x, sel_l0, sel_l1, sel_l2, sel_l3):
    n = x.shape[0]
    grid = (n // _NB,)
    full = lambda shape: pl.BlockSpec(shape, lambda i: (0, 0))
    return pl.pallas_call(
        _tc_body,
        grid=grid,
        in_specs=[
            pl.BlockSpec((_NB, 512), lambda i: (i, 0)),
            full((256, 512)),
            full((128, 256)),
            full((256, 128)),
            full((512, 256)),
        ],
        out_specs=pl.BlockSpec((_NB, 512), lambda i: (i, 0)),
        out_shape=jax.ShapeDtypeStruct((n, 512), jnp.float32),
    )(x, sel_l0, sel_l1, sel_l2, sel_l3)
